# baseline (speedup 1.0000x reference)
import functools

import jax
import jax.numpy as jnp
from jax import lax
from jax.experimental import pallas as pl
from jax.experimental.pallas import tpu as pltpu
from jax.experimental.pallas import tpu_sc as plsc

B = 8
TA = 1024
TB = 1024
T = TA + TB
VA, VB = 2000, 500
NV = VA + VB
DEMB, D, H = 64, 256, 256
G = 3 * H
BT = 128
NB = T // BT
ROWS = T * B


def _tables_body(emb_a_ref, emb_b_ref, wa_ref, wb_ref,
                 ba_ref, bb_ref, ta_ref, tb_ref):
    ta_ref[...] = jnp.dot(emb_a_ref[...], wa_ref[...],
                          preferred_element_type=jnp.float32) + ba_ref[...]
    tb_ref[...] = jnp.dot(emb_b_ref[...], wb_ref[...],
                          preferred_element_type=jnp.float32) + bb_ref[...]


def _make_tables(emb_a, emb_b, proj_a_W, proj_b_W, proj_a_b, proj_b_b):
    return pl.pallas_call(
        _tables_body,
        out_shape=[
            jax.ShapeDtypeStruct((VA, D), jnp.float32),
            jax.ShapeDtypeStruct((VB, D), jnp.float32),
        ],
    )(emb_a, emb_b, proj_a_W, proj_b_W,
      proj_a_b.reshape(1, D), proj_b_b.reshape(1, D))


def _sort_body(at_ref, bt_ref, ac_ref, bc_ref, la_ref, lb_ref, gid_ref):
    iota_t = lax.broadcasted_iota(jnp.int32, (B, TA), 1)
    va = iota_t < la_ref[:, :1]
    vb = iota_t < lb_ref[:, :1]
    ta = jnp.where(va, at_ref[...], 0.0)
    tb = jnp.where(vb, bt_ref[...], 0.0)
    key = jnp.concatenate([ta, tb], axis=1)
    key = jnp.where(key == 0.0, jnp.inf, key)
    ga = jnp.where(va, ac_ref[...], 0)
    gb = jnp.where(vb, bc_ref[...], 0) + VA
    gid = jnp.concatenate([ga, gb], axis=1)
    idx = lax.broadcasted_iota(jnp.int32, (B, T), 1)
    pos = lax.broadcasted_iota(jnp.int32, (1, T), 1)

    k = 2
    while k <= T:
        j = k // 2
        while j >= 1:
            i_low = (pos & j) == 0
            up = (pos & k) == 0
            keep_min = i_low == up
            key_p = jnp.where(i_low, jnp.roll(key, -j, axis=1),
                              jnp.roll(key, j, axis=1))
            idx_p = jnp.where(i_low, jnp.roll(idx, -j, axis=1),
                              jnp.roll(idx, j, axis=1))
            gid_p = jnp.where(i_low, jnp.roll(gid, -j, axis=1),
                              jnp.roll(gid, j, axis=1))
            less = (key < key_p) | ((key == key_p) & (idx < idx_p))
            take_self = less == keep_min
            key = jnp.where(take_self, key, key_p)
            idx = jnp.where(take_self, idx, idx_p)
            gid = jnp.where(take_self, gid, gid_p)
            j //= 2
        k *= 2
    gid_ref[...] = gid


def _sorted_gids(a_time, b_time, a_cat, b_cat, la2, lb2):
    return pl.pallas_call(
        _sort_body,
        out_shape=jax.ShapeDtypeStruct((B, T), jnp.int32),
    )(a_time, b_time, a_cat, b_cat, la2, lb2)


_NBUF = 8
_NSTAGE = 8
SROWS = ROWS // _NSTAGE
ST = T // _NSTAGE


@functools.lru_cache(maxsize=None)
def _make_sc_gather(stage):
    info = plsc.get_sparse_core_info()
    NC, NS = info.num_cores, info.num_subcores
    NW = NC * NS
    b_per_w = SROWS // NW
    CH = 32
    NCH = b_per_w // CH
    mesh = plsc.VectorSubcoreMesh(core_axis_name="c", subcore_axis_name="s")

    @functools.partial(
        pl.kernel, mesh=mesh,
        out_type=jax.ShapeDtypeStruct((SROWS, D), jnp.float32),
        scratch_types=[pltpu.VMEM((b_per_w,), jnp.int32)]
        + [pltpu.VMEM((CH, D), jnp.float32)] * _NBUF
        + [pltpu.SemaphoreType.DMA] * (2 * _NBUF),
    )
    def gather_k(table_hbm, idx_hbm, out_hbm, idx_v, *rest):
        bufs = rest[:_NBUF]
        gsems = rest[_NBUF:2 * _NBUF]
        osems = rest[2 * _NBUF:]
        wid = lax.axis_index("s") * NC + lax.axis_index("c")
        base = wid * b_per_w
        pltpu.sync_copy(
            idx_hbm.at[pl.ds(stage * SROWS + base, b_per_w)], idx_v)

        def start_gather(c):
            return pltpu.async_copy(
                table_hbm.at[idx_v.at[pl.ds(c * CH, CH)]],
                bufs[c % _NBUF], gsems[c % _NBUF])

        gathers = [None] * _NBUF
        outs = [None] * _NBUF
        for c in range(min(_NBUF, NCH)):
            gathers[c % _NBUF] = start_gather(c)
        for c in range(NCH):
            i = c % _NBUF
            gathers[i].wait()
            outs[i] = pltpu.async_copy(
                bufs[i], out_hbm.at[pl.ds(base + c * CH, CH)], osems[i])
            n = c + _NBUF
            if n < NCH:
                outs[i].wait()
                gathers[i] = start_gather(n)
        for c in range(max(NCH - _NBUF, 0), NCH):
            outs[c % _NBUF].wait()

    return gather_k


NBS = ST // BT


def _gru_body(stage, la_ref, lb_ref, x_ref, wi_ref, wh_ref, bi_ref, bh_ref,
              h0_ref, acc0_ref, hout_ref, aout_ref, h_ref, acc_ref, gi_ref):
    i = pl.program_id(0)

    @pl.when(i == 0)
    def _():
        h_ref[...] = h0_ref[...]
        acc_ref[...] = acc0_ref[...]

    lens = la_ref[...] + lb_ref[...]
    maxlen = jnp.max(lens)
    if stage == 0:
        maxlen = jnp.maximum(maxlen, 1)
    base = stage * ST + i * BT
    steps = jnp.clip(maxlen - base, 0, BT)

    @pl.when(steps > 0)
    def _():
        xb = x_ref[...].reshape(BT * B, D)
        gi_ref[...] = jnp.dot(xb, wi_ref[...],
                              preferred_element_type=jnp.float32) + bi_ref[...]
        target = jnp.clip(lens[:, :1] - 1, 0, T - 1)
        wh = wh_ref[...].astype(jnp.bfloat16)
        bh = bh_ref[...]

        def body(t, carry):
            h, acc = carry
            gix = gi_ref[pl.ds(t * B, B)]
            gh = jnp.dot(h.astype(jnp.bfloat16), wh,
                         preferred_element_type=jnp.float32) + bh
            r = jax.nn.sigmoid(gix[:, :H] + gh[:, :H])
            z = jax.nn.sigmoid(gix[:, H:2 * H] + gh[:, H:2 * H])
            n = jnp.tanh(gix[:, 2 * H:] + r * gh[:, 2 * H:])
            h = (1.0 - z) * n + z * h
            acc = jnp.where(target == base + t, h, acc)
            return h, acc

        h, acc = lax.fori_loop(0, steps, body, (h_ref[...], acc_ref[...]))
        h_ref[...] = h
        acc_ref[...] = acc

    @pl.when(i == NBS - 1)
    def _():
        hout_ref[...] = h_ref[...]
        aout_ref[...] = acc_ref[...]


def _gru_stage(stage, la2, lb2, x3, gru_Wi, gru_Wh, bi2, bh2, h0, acc0):
    return pl.pallas_call(
        functools.partial(_gru_body, stage),
        grid=(NBS,),
        in_specs=[
            pl.BlockSpec((B, 128), lambda i: (0, 0)),
            pl.BlockSpec((B, 128), lambda i: (0, 0)),
            pl.BlockSpec((BT, B, D), lambda i: (i, 0, 0)),
            pl.BlockSpec((D, G), lambda i: (0, 0)),
            pl.BlockSpec((H, G), lambda i: (0, 0)),
            pl.BlockSpec((1, G), lambda i: (0, 0)),
            pl.BlockSpec((1, G), lambda i: (0, 0)),
            pl.BlockSpec((B, H), lambda i: (0, 0)),
            pl.BlockSpec((B, H), lambda i: (0, 0)),
        ],
        out_specs=[
            pl.BlockSpec((B, H), lambda i: (0, 0)),
            pl.BlockSpec((B, H), lambda i: (0, 0)),
        ],
        out_shape=[
            jax.ShapeDtypeStruct((B, H), jnp.float32),
            jax.ShapeDtypeStruct((B, H), jnp.float32),
        ],
        scratch_shapes=[
            pltpu.VMEM((B, H), jnp.float32),
            pltpu.VMEM((B, H), jnp.float32),
            pltpu.VMEM((BT * B, G), jnp.float32),
        ],
    )(la2, lb2, x3, gru_Wi, gru_Wh, bi2, bh2, h0, acc0)


def kernel(a_cat, a_time, a_seq_lens, b_cat, b_time, b_seq_lens,
           emb_a, emb_b, proj_a_W, proj_a_b, proj_b_W, proj_b_b,
           gru_Wi, gru_Wh, gru_bi, gru_bh):
    ta_tab, tb_tab = _make_tables(emb_a, emb_b, proj_a_W, proj_b_W,
                                  proj_a_b, proj_b_b)
    table = jnp.concatenate([ta_tab, tb_tab], axis=0)

    la2 = jnp.broadcast_to(a_seq_lens.astype(jnp.int32)[:, None], (B, 128))
    lb2 = jnp.broadcast_to(b_seq_lens.astype(jnp.int32)[:, None], (B, 128))

    gid = _sorted_gids(a_time, b_time, a_cat.astype(jnp.int32),
                       b_cat.astype(jnp.int32), la2, lb2)
    gid_flat = gid.T.reshape(ROWS)

    bi2 = gru_bi.reshape(1, G)
    bh2 = gru_bh.reshape(1, G)
    h = jnp.zeros((B, H), jnp.float32)
    acc = jnp.zeros((B, H), jnp.float32)
    xs = [_make_sc_gather(s)(table, gid_flat) for s in range(_NSTAGE)]
    for s in range(_NSTAGE):
        x3 = xs[s].reshape(ST, B, D)
        h, acc = _gru_stage(s, la2, lb2, x3, gru_Wi, gru_Wh, bi2, bh2, h, acc)
    return acc


# speedup vs baseline: 1.0409x; 1.0409x over previous
import functools

import jax
import jax.numpy as jnp
from jax import lax
from jax.experimental import pallas as pl
from jax.experimental.pallas import tpu as pltpu
from jax.experimental.pallas import tpu_sc as plsc

B = 8
TA = 1024
TB = 1024
T = TA + TB
VA, VB = 2000, 500
NV = VA + VB
DEMB, D, H = 64, 256, 256
G = 3 * H
BT = 128
NB = T // BT
ROWS = T * B


def _tables_body(emb_a_ref, emb_b_ref, wa_ref, wb_ref,
                 ba_ref, bb_ref, ta_ref, tb_ref):
    ta_ref[...] = jnp.dot(emb_a_ref[...], wa_ref[...],
                          preferred_element_type=jnp.float32) + ba_ref[...]
    tb_ref[...] = jnp.dot(emb_b_ref[...], wb_ref[...],
                          preferred_element_type=jnp.float32) + bb_ref[...]


def _make_tables(emb_a, emb_b, proj_a_W, proj_b_W, proj_a_b, proj_b_b):
    return pl.pallas_call(
        _tables_body,
        out_shape=[
            jax.ShapeDtypeStruct((VA, D), jnp.float32),
            jax.ShapeDtypeStruct((VB, D), jnp.float32),
        ],
    )(emb_a, emb_b, proj_a_W, proj_b_W,
      proj_a_b.reshape(1, D), proj_b_b.reshape(1, D))


def _sort_body(at_ref, bt_ref, ac_ref, bc_ref, la_ref, lb_ref, gid_ref):
    iota_t = lax.broadcasted_iota(jnp.int32, (B, TA), 1)
    va = iota_t < la_ref[:, :1]
    vb = iota_t < lb_ref[:, :1]
    ta = jnp.where(va, at_ref[...], 0.0)
    tb = jnp.where(vb, bt_ref[...], 0.0)
    key = jnp.concatenate([ta, tb], axis=1)
    key = jnp.where(key == 0.0, jnp.inf, key)
    ga = jnp.where(va, ac_ref[...], 0)
    gb = jnp.where(vb, bc_ref[...], 0) + VA
    gid = jnp.concatenate([ga, gb], axis=1)
    idx = lax.broadcasted_iota(jnp.int32, (B, T), 1)
    pos = lax.broadcasted_iota(jnp.int32, (1, T), 1)

    k = 2
    while k <= T:
        j = k // 2
        while j >= 1:
            i_low = (pos & j) == 0
            up = (pos & k) == 0
            keep_min = i_low == up
            key_p = jnp.where(i_low, jnp.roll(key, -j, axis=1),
                              jnp.roll(key, j, axis=1))
            idx_p = jnp.where(i_low, jnp.roll(idx, -j, axis=1),
                              jnp.roll(idx, j, axis=1))
            gid_p = jnp.where(i_low, jnp.roll(gid, -j, axis=1),
                              jnp.roll(gid, j, axis=1))
            less = (key < key_p) | ((key == key_p) & (idx < idx_p))
            take_self = less == keep_min
            key = jnp.where(take_self, key, key_p)
            idx = jnp.where(take_self, idx, idx_p)
            gid = jnp.where(take_self, gid, gid_p)
            j //= 2
        k *= 2
    gid_ref[...] = gid


def _sorted_gids(a_time, b_time, a_cat, b_cat, la2, lb2):
    return pl.pallas_call(
        _sort_body,
        out_shape=jax.ShapeDtypeStruct((B, T), jnp.int32),
    )(a_time, b_time, a_cat, b_cat, la2, lb2)


_NBUF = 8
_NSTAGE = 4
SROWS = ROWS // _NSTAGE
ST = T // _NSTAGE


@functools.lru_cache(maxsize=None)
def _make_sc_gather(stage):
    info = plsc.get_sparse_core_info()
    NC, NS = info.num_cores, info.num_subcores
    NW = NC * NS
    b_per_w = SROWS // NW
    CH = 32
    NCH = b_per_w // CH
    mesh = plsc.VectorSubcoreMesh(core_axis_name="c", subcore_axis_name="s")

    @functools.partial(
        pl.kernel, mesh=mesh,
        out_type=jax.ShapeDtypeStruct((SROWS, D), jnp.float32),
        scratch_types=[pltpu.VMEM((b_per_w,), jnp.int32)]
        + [pltpu.VMEM((CH, D), jnp.float32)] * _NBUF
        + [pltpu.SemaphoreType.DMA] * (2 * _NBUF),
    )
    def gather_k(table_hbm, idx_hbm, out_hbm, idx_v, *rest):
        bufs = rest[:_NBUF]
        gsems = rest[_NBUF:2 * _NBUF]
        osems = rest[2 * _NBUF:]
        wid = lax.axis_index("s") * NC + lax.axis_index("c")
        base = wid * b_per_w
        pltpu.sync_copy(
            idx_hbm.at[pl.ds(stage * SROWS + base, b_per_w)], idx_v)

        def start_gather(c):
            return pltpu.async_copy(
                table_hbm.at[idx_v.at[pl.ds(c * CH, CH)]],
                bufs[c % _NBUF], gsems[c % _NBUF])

        gathers = [None] * _NBUF
        outs = [None] * _NBUF
        for c in range(min(_NBUF, NCH)):
            gathers[c % _NBUF] = start_gather(c)
        for c in range(NCH):
            i = c % _NBUF
            gathers[i].wait()
            outs[i] = pltpu.async_copy(
                bufs[i], out_hbm.at[pl.ds(base + c * CH, CH)], osems[i])
            n = c + _NBUF
            if n < NCH:
                outs[i].wait()
                gathers[i] = start_gather(n)
        for c in range(max(NCH - _NBUF, 0), NCH):
            outs[c % _NBUF].wait()

    return gather_k


NBS = ST // BT


def _gru_body(stage, la_ref, lb_ref, x_ref, wi_ref, wh_ref, bi_ref, bh_ref,
              h0_ref, acc0_ref, hout_ref, aout_ref, h_ref, acc_ref, gi_ref):
    i = pl.program_id(0)

    @pl.when(i == 0)
    def _():
        h_ref[...] = h0_ref[...]
        acc_ref[...] = acc0_ref[...]

    lens = la_ref[...] + lb_ref[...]
    maxlen = jnp.max(lens)
    if stage == 0:
        maxlen = jnp.maximum(maxlen, 1)
    base = stage * ST + i * BT
    steps = jnp.clip(maxlen - base, 0, BT)

    @pl.when(steps > 0)
    def _():
        xb = x_ref[...].reshape(BT * B, D)
        gi_ref[...] = jnp.dot(xb, wi_ref[...],
                              preferred_element_type=jnp.float32) + bi_ref[...]
        target = jnp.clip(lens[:, :1] - 1, 0, T - 1)
        wh = wh_ref[...].astype(jnp.bfloat16)
        bh = bh_ref[...]
        HB = B // 2

        def half_step(hx, gix, gh):
            r = jax.nn.sigmoid(gix[:, :H] + gh[:, :H])
            z = jax.nn.sigmoid(gix[:, H:2 * H] + gh[:, H:2 * H])
            n = jnp.tanh(gix[:, 2 * H:] + r * gh[:, 2 * H:])
            return (1.0 - z) * n + z * hx

        def body(t, carry):
            ha, hb, aa, ab = carry
            gix = gi_ref[pl.ds(t * B, B)]
            gha = jnp.dot(ha.astype(jnp.bfloat16), wh,
                          preferred_element_type=jnp.float32) + bh
            ghb = jnp.dot(hb.astype(jnp.bfloat16), wh,
                          preferred_element_type=jnp.float32) + bh
            ha = half_step(ha, gix[:HB], gha)
            hb = half_step(hb, gix[HB:], ghb)
            cap = target == base + t
            aa = jnp.where(cap[:HB], ha, aa)
            ab = jnp.where(cap[HB:], hb, ab)
            return ha, hb, aa, ab

        ha, hb, aa, ab = lax.fori_loop(
            0, steps, body,
            (h_ref[:HB], h_ref[HB:], acc_ref[:HB], acc_ref[HB:]))
        h_ref[...] = jnp.concatenate([ha, hb], axis=0)
        acc_ref[...] = jnp.concatenate([aa, ab], axis=0)

    @pl.when(i == NBS - 1)
    def _():
        hout_ref[...] = h_ref[...]
        aout_ref[...] = acc_ref[...]


def _gru_stage(stage, la2, lb2, x3, gru_Wi, gru_Wh, bi2, bh2, h0, acc0):
    return pl.pallas_call(
        functools.partial(_gru_body, stage),
        grid=(NBS,),
        in_specs=[
            pl.BlockSpec((B, 128), lambda i: (0, 0)),
            pl.BlockSpec((B, 128), lambda i: (0, 0)),
            pl.BlockSpec((BT, B, D), lambda i: (i, 0, 0)),
            pl.BlockSpec((D, G), lambda i: (0, 0)),
            pl.BlockSpec((H, G), lambda i: (0, 0)),
            pl.BlockSpec((1, G), lambda i: (0, 0)),
            pl.BlockSpec((1, G), lambda i: (0, 0)),
            pl.BlockSpec((B, H), lambda i: (0, 0)),
            pl.BlockSpec((B, H), lambda i: (0, 0)),
        ],
        out_specs=[
            pl.BlockSpec((B, H), lambda i: (0, 0)),
            pl.BlockSpec((B, H), lambda i: (0, 0)),
        ],
        out_shape=[
            jax.ShapeDtypeStruct((B, H), jnp.float32),
            jax.ShapeDtypeStruct((B, H), jnp.float32),
        ],
        scratch_shapes=[
            pltpu.VMEM((B, H), jnp.float32),
            pltpu.VMEM((B, H), jnp.float32),
            pltpu.VMEM((BT * B, G), jnp.float32),
        ],
    )(la2, lb2, x3, gru_Wi, gru_Wh, bi2, bh2, h0, acc0)


def kernel(a_cat, a_time, a_seq_lens, b_cat, b_time, b_seq_lens,
           emb_a, emb_b, proj_a_W, proj_a_b, proj_b_W, proj_b_b,
           gru_Wi, gru_Wh, gru_bi, gru_bh):
    ta_tab, tb_tab = _make_tables(emb_a, emb_b, proj_a_W, proj_b_W,
                                  proj_a_b, proj_b_b)
    table = jnp.concatenate([ta_tab, tb_tab], axis=0)

    la2 = jnp.broadcast_to(a_seq_lens.astype(jnp.int32)[:, None], (B, 128))
    lb2 = jnp.broadcast_to(b_seq_lens.astype(jnp.int32)[:, None], (B, 128))

    gid = _sorted_gids(a_time, b_time, a_cat.astype(jnp.int32),
                       b_cat.astype(jnp.int32), la2, lb2)
    gid_flat = gid.T.reshape(ROWS)

    bi2 = gru_bi.reshape(1, G)
    bh2 = gru_bh.reshape(1, G)
    h = jnp.zeros((B, H), jnp.float32)
    acc = jnp.zeros((B, H), jnp.float32)
    xs = [_make_sc_gather(s)(table, gid_flat) for s in range(_NSTAGE)]
    for s in range(_NSTAGE):
        x3 = xs[s].reshape(ST, B, D)
        h, acc = _gru_stage(s, la2, lb2, x3, gru_Wi, gru_Wh, bi2, bh2, h, acc)
    return acc


# speedup vs baseline: 1.2260x; 1.1778x over previous
import functools

import jax
import jax.numpy as jnp
from jax import lax
from jax.experimental import pallas as pl
from jax.experimental.pallas import tpu as pltpu
from jax.experimental.pallas import tpu_sc as plsc

B = 8
TA = 1024
TB = 1024
T = TA + TB
VA, VB = 2000, 500
NV = VA + VB
DEMB, D, H = 64, 256, 256
G = 3 * H
BT = 128
NB = T // BT
ROWS = T * B


def _tables_body(emb_a_ref, emb_b_ref, wa_ref, wb_ref,
                 ba_ref, bb_ref, ta_ref, tb_ref):
    ta_ref[...] = jnp.dot(emb_a_ref[...], wa_ref[...],
                          preferred_element_type=jnp.float32) + ba_ref[...]
    tb_ref[...] = jnp.dot(emb_b_ref[...], wb_ref[...],
                          preferred_element_type=jnp.float32) + bb_ref[...]


def _make_tables(emb_a, emb_b, proj_a_W, proj_b_W, proj_a_b, proj_b_b):
    return pl.pallas_call(
        _tables_body,
        out_shape=[
            jax.ShapeDtypeStruct((VA, D), jnp.float32),
            jax.ShapeDtypeStruct((VB, D), jnp.float32),
        ],
    )(emb_a, emb_b, proj_a_W, proj_b_W,
      proj_a_b.reshape(1, D), proj_b_b.reshape(1, D))


def _sort_body(at_ref, bt_ref, ac_ref, bc_ref, la_ref, lb_ref, gid_ref):
    iota_t = lax.broadcasted_iota(jnp.int32, (B, TA), 1)
    va = iota_t < la_ref[:, :1]
    vb = iota_t < lb_ref[:, :1]
    ta = jnp.where(va, at_ref[...], 0.0)
    tb = jnp.where(vb, bt_ref[...], 0.0)
    key = jnp.concatenate([ta, tb], axis=1)
    key = jnp.where(key == 0.0, jnp.inf, key)
    ga = jnp.where(va, ac_ref[...], 0)
    gb = jnp.where(vb, bc_ref[...], 0) + VA
    gid = jnp.concatenate([ga, gb], axis=1)
    idx = lax.broadcasted_iota(jnp.int32, (B, T), 1)
    pos = lax.broadcasted_iota(jnp.int32, (1, T), 1)

    k = 2
    while k <= T:
        j = k // 2
        while j >= 1:
            i_low = (pos & j) == 0
            up = (pos & k) == 0
            keep_min = i_low == up
            key_p = jnp.where(i_low, jnp.roll(key, -j, axis=1),
                              jnp.roll(key, j, axis=1))
            idx_p = jnp.where(i_low, jnp.roll(idx, -j, axis=1),
                              jnp.roll(idx, j, axis=1))
            gid_p = jnp.where(i_low, jnp.roll(gid, -j, axis=1),
                              jnp.roll(gid, j, axis=1))
            less = (key < key_p) | ((key == key_p) & (idx < idx_p))
            take_self = less == keep_min
            key = jnp.where(take_self, key, key_p)
            idx = jnp.where(take_self, idx, idx_p)
            gid = jnp.where(take_self, gid, gid_p)
            j //= 2
        k *= 2
    gid_ref[...] = gid


def _sorted_gids(a_time, b_time, a_cat, b_cat, la2, lb2):
    return pl.pallas_call(
        _sort_body,
        out_shape=jax.ShapeDtypeStruct((B, T), jnp.int32),
    )(a_time, b_time, a_cat, b_cat, la2, lb2)


_NBUF = 8
_NSTAGE = 4
SROWS = ROWS // _NSTAGE
ST = T // _NSTAGE


@functools.lru_cache(maxsize=None)
def _make_sc_gather(stage):
    info = plsc.get_sparse_core_info()
    NC, NS = info.num_cores, info.num_subcores
    NW = NC * NS
    b_per_w = SROWS // NW
    CH = 32
    NCH = b_per_w // CH
    mesh = plsc.VectorSubcoreMesh(core_axis_name="c", subcore_axis_name="s")

    @functools.partial(
        pl.kernel, mesh=mesh,
        out_type=jax.ShapeDtypeStruct((SROWS, D), jnp.float32),
        scratch_types=[pltpu.VMEM((b_per_w,), jnp.int32)]
        + [pltpu.VMEM((CH, D), jnp.float32)] * _NBUF
        + [pltpu.SemaphoreType.DMA] * (2 * _NBUF),
    )
    def gather_k(table_hbm, idx_hbm, out_hbm, idx_v, *rest):
        bufs = rest[:_NBUF]
        gsems = rest[_NBUF:2 * _NBUF]
        osems = rest[2 * _NBUF:]
        wid = lax.axis_index("s") * NC + lax.axis_index("c")
        base = wid * b_per_w
        pltpu.sync_copy(
            idx_hbm.at[pl.ds(stage * SROWS + base, b_per_w)], idx_v)

        def start_gather(c):
            return pltpu.async_copy(
                table_hbm.at[idx_v.at[pl.ds(c * CH, CH)]],
                bufs[c % _NBUF], gsems[c % _NBUF])

        gathers = [None] * _NBUF
        outs = [None] * _NBUF
        for c in range(min(_NBUF, NCH)):
            gathers[c % _NBUF] = start_gather(c)
        for c in range(NCH):
            i = c % _NBUF
            gathers[i].wait()
            outs[i] = pltpu.async_copy(
                bufs[i], out_hbm.at[pl.ds(base + c * CH, CH)], osems[i])
            n = c + _NBUF
            if n < NCH:
                outs[i].wait()
                gathers[i] = start_gather(n)
        for c in range(max(NCH - _NBUF, 0), NCH):
            outs[c % _NBUF].wait()

    return gather_k


NBS = ST // BT


def _gru_body(stage, la_ref, lb_ref, x_ref, wi_ref, wh_ref, bi_ref, bh_ref,
              h0_ref, acc0_ref, hout_ref, aout_ref, h_ref, acc_ref, gi_ref):
    i = pl.program_id(0)

    @pl.when(i == 0)
    def _():
        h_ref[...] = h0_ref[...]
        acc_ref[...] = acc0_ref[...]

    lens = la_ref[...] + lb_ref[...]
    maxlen = jnp.max(lens)
    if stage == 0:
        maxlen = jnp.maximum(maxlen, 1)
    base = stage * ST + i * BT
    steps = jnp.clip(maxlen - base, 0, BT)

    @pl.when(steps > 0)
    def _():
        xb = x_ref[...].reshape(BT * B, D)
        gi_ref[...] = jnp.dot(xb, wi_ref[...],
                              preferred_element_type=jnp.float32) + bi_ref[...]
        target = jnp.clip(lens[:, :1] - 1, 0, T - 1)
        wh = wh_ref[...]
        bh = bh_ref[...]

        def body(t, carry):
            h, acc = carry
            gix = gi_ref[pl.ds(t * B, B)]
            gh = jnp.dot(h.astype(jnp.bfloat16), wh,
                         preferred_element_type=jnp.float32) + bh
            r = jax.nn.sigmoid(gix[:, :H] + gh[:, :H])
            z = jax.nn.sigmoid(gix[:, H:2 * H] + gh[:, H:2 * H])
            n = jnp.tanh(gix[:, 2 * H:] + r * gh[:, 2 * H:])
            h = (1.0 - z) * n + z * h
            acc = jnp.where(target == base + t, h, acc)
            return h, acc

        h, acc = lax.fori_loop(0, steps, body, (h_ref[...], acc_ref[...]))
        h_ref[...] = h
        acc_ref[...] = acc

    @pl.when(i == NBS - 1)
    def _():
        hout_ref[...] = h_ref[...]
        aout_ref[...] = acc_ref[...]


def _gru_stage(stage, la2, lb2, x3, gru_Wi, gru_Wh, bi2, bh2, h0, acc0):
    return pl.pallas_call(
        functools.partial(_gru_body, stage),
        grid=(NBS,),
        in_specs=[
            pl.BlockSpec((B, 128), lambda i: (0, 0)),
            pl.BlockSpec((B, 128), lambda i: (0, 0)),
            pl.BlockSpec((BT, B, D), lambda i: (i, 0, 0)),
            pl.BlockSpec((D, G), lambda i: (0, 0)),
            pl.BlockSpec((H, G), lambda i: (0, 0)),
            pl.BlockSpec((1, G), lambda i: (0, 0)),
            pl.BlockSpec((1, G), lambda i: (0, 0)),
            pl.BlockSpec((B, H), lambda i: (0, 0)),
            pl.BlockSpec((B, H), lambda i: (0, 0)),
        ],
        out_specs=[
            pl.BlockSpec((B, H), lambda i: (0, 0)),
            pl.BlockSpec((B, H), lambda i: (0, 0)),
        ],
        out_shape=[
            jax.ShapeDtypeStruct((B, H), jnp.float32),
            jax.ShapeDtypeStruct((B, H), jnp.float32),
        ],
        scratch_shapes=[
            pltpu.VMEM((B, H), jnp.float32),
            pltpu.VMEM((B, H), jnp.float32),
            pltpu.VMEM((BT * B, G), jnp.float32),
        ],
    )(la2, lb2, x3, gru_Wi, gru_Wh, bi2, bh2, h0, acc0)


def kernel(a_cat, a_time, a_seq_lens, b_cat, b_time, b_seq_lens,
           emb_a, emb_b, proj_a_W, proj_a_b, proj_b_W, proj_b_b,
           gru_Wi, gru_Wh, gru_bi, gru_bh):
    ta_tab, tb_tab = _make_tables(emb_a, emb_b, proj_a_W, proj_b_W,
                                  proj_a_b, proj_b_b)
    table = jnp.concatenate([ta_tab, tb_tab], axis=0)

    la2 = jnp.broadcast_to(a_seq_lens.astype(jnp.int32)[:, None], (B, 128))
    lb2 = jnp.broadcast_to(b_seq_lens.astype(jnp.int32)[:, None], (B, 128))

    gid = _sorted_gids(a_time, b_time, a_cat.astype(jnp.int32),
                       b_cat.astype(jnp.int32), la2, lb2)
    gid_flat = gid.T.reshape(ROWS)

    bi2 = gru_bi.reshape(1, G)
    bh2 = gru_bh.reshape(1, G)
    wh16 = gru_Wh.astype(jnp.bfloat16)
    h = jnp.zeros((B, H), jnp.float32)
    acc = jnp.zeros((B, H), jnp.float32)
    xs = [_make_sc_gather(s)(table, gid_flat) for s in range(_NSTAGE)]
    for s in range(_NSTAGE):
        x3 = xs[s].reshape(ST, B, D)
        h, acc = _gru_stage(s, la2, lb2, x3, gru_Wi, wh16, bi2, bh2, h, acc)
    return acc
